# baseline (speedup 1.0000x reference)
import functools

import jax
import jax.numpy as jnp
from jax.experimental import pallas as pl
from jax.experimental.pallas import tpu as pltpu

TN = 512
TM = 512


def _chamfer_body(x1_ref, x2t_ref, d1_ref, d2_ref):
    m = pl.program_id(1)
    n = pl.program_id(2)

    x1 = x1_ref[0]
    x2t = x2t_ref[0]

    acc = None
    for k in range(3):
        diff = x1[:, k : k + 1] - x2t[k : k + 1, :]
        sq = diff * diff
        acc = sq if acc is None else acc + sq

    row_min = jnp.min(acc, axis=1)
    col_min = jnp.min(acc, axis=0)

    sl = pl.ds(n * TN, TN)

    @pl.when(m == 0)
    def _():
        d1_ref[0, 0, sl] = row_min

    @pl.when(m > 0)
    def _():
        d1_ref[0, 0, sl] = jnp.minimum(d1_ref[0, 0, sl], row_min)

    @pl.when(n == 0)
    def _():
        d2_ref[0, 0, :] = col_min

    @pl.when(n > 0)
    def _():
        d2_ref[0, 0, :] = jnp.minimum(d2_ref[0, 0, :], col_min)


@jax.jit
def kernel(xyz1, xyz2):
    B, N, _ = xyz1.shape
    _, M, _ = xyz2.shape
    x2t = jnp.transpose(xyz2, (0, 2, 1))

    grid = (B, M // TM, N // TN)
    dist1, dist2 = pl.pallas_call(
        _chamfer_body,
        grid=grid,
        in_specs=[
            pl.BlockSpec((1, TN, 3), lambda b, m, n: (b, n, 0)),
            pl.BlockSpec((1, 3, TM), lambda b, m, n: (b, 0, m)),
        ],
        out_specs=[
            pl.BlockSpec((1, 1, N), lambda b, m, n: (b, 0, 0)),
            pl.BlockSpec((1, 1, TM), lambda b, m, n: (b, 0, m)),
        ],
        out_shape=[
            jax.ShapeDtypeStruct((B, 1, N), jnp.float32),
            jax.ShapeDtypeStruct((B, 1, M), jnp.float32),
        ],
        compiler_params=pltpu.CompilerParams(
            dimension_semantics=("arbitrary", "arbitrary", "arbitrary"),
        ),
    )(xyz1, x2t)
    return (dist1[:, 0, :], dist2[:, 0, :])


# speedup vs baseline: 1.1541x; 1.1541x over previous
import jax
import jax.numpy as jnp
from jax.experimental import pallas as pl
from jax.experimental.pallas import tpu as pltpu

TN = 512
TM = 512


def _chamfer_body(x1_ref, x2t_ref, d1_ref, d2_ref, part_ref):
    m = pl.program_id(1)
    n = pl.program_id(2)
    num_m = pl.num_programs(1)

    x1 = x1_ref[0]
    x2t = x2t_ref[0]

    acc = None
    for k in range(3):
        diff = x1[:, k : k + 1] - x2t[k : k + 1, :]
        sq = diff * diff
        acc = sq if acc is None else acc + sq

    part = acc[:, 0:128]
    for j in range(1, TM // 128):
        part = jnp.minimum(part, acc[:, j * 128 : (j + 1) * 128])

    col_min = jnp.min(acc, axis=0)

    sl = pl.ds(n * TN, TN)

    @pl.when(m == 0)
    def _():
        part_ref[sl, :] = part

    @pl.when(m > 0)
    def _():
        part_ref[sl, :] = jnp.minimum(part_ref[sl, :], part)

    @pl.when(m == num_m - 1)
    def _():
        d1_ref[0, 0, sl] = jnp.min(part_ref[sl, :], axis=1)

    @pl.when(n == 0)
    def _():
        d2_ref[0, 0, :] = col_min

    @pl.when(n > 0)
    def _():
        d2_ref[0, 0, :] = jnp.minimum(d2_ref[0, 0, :], col_min)


@jax.jit
def kernel(xyz1, xyz2):
    B, N, _ = xyz1.shape
    _, M, _ = xyz2.shape
    x2t = jnp.transpose(xyz2, (0, 2, 1))

    grid = (B, M // TM, N // TN)
    dist1, dist2 = pl.pallas_call(
        _chamfer_body,
        grid=grid,
        in_specs=[
            pl.BlockSpec((1, TN, 3), lambda b, m, n: (b, n, 0)),
            pl.BlockSpec((1, 3, TM), lambda b, m, n: (b, 0, m)),
        ],
        out_specs=[
            pl.BlockSpec((1, 1, N), lambda b, m, n: (b, 0, 0)),
            pl.BlockSpec((1, 1, TM), lambda b, m, n: (b, 0, m)),
        ],
        out_shape=[
            jax.ShapeDtypeStruct((B, 1, N), jnp.float32),
            jax.ShapeDtypeStruct((B, 1, M), jnp.float32),
        ],
        scratch_shapes=[pltpu.VMEM((N, 128), jnp.float32)],
        compiler_params=pltpu.CompilerParams(
            dimension_semantics=("arbitrary", "arbitrary", "arbitrary"),
        ),
    )(xyz1, x2t)
    return (dist1[:, 0, :], dist2[:, 0, :])


# speedup vs baseline: 1.8152x; 1.5729x over previous
import jax
import jax.numpy as jnp
from jax.experimental import pallas as pl
from jax.experimental.pallas import tpu as pltpu

TN = 512
TM = 1024


def _chamfer_body(x1_ref, x2t_ref, d1_ref, d2_ref, d1s_ref, d2s_ref):
    n = pl.program_id(1)
    m = pl.program_id(2)
    num_n = pl.num_programs(1)
    num_m = pl.num_programs(2)
    J = TM // 128
    R = TN // 8

    @pl.when(m == 0)
    def _():
        d1s_ref[...] = jnp.full((TN, 128), jnp.inf, jnp.float32)

    @pl.when((n == 0) & (m == 0))
    def _():
        d2s_ref[...] = jnp.full(d2s_ref.shape, jnp.inf, jnp.float32)

    t = [
        [
            jnp.broadcast_to(
                x2t_ref[0, k : k + 1, pl.ds(j * 128, 128)], (8, 128)
            )
            for k in range(3)
        ]
        for j in range(J)
    ]

    colacc = [None] * J
    for r in range(R):
        rs = pl.ds(r * 8, 8)
        a = [
            jnp.broadcast_to(x1_ref[0, rs, k : k + 1], (8, 128))
            for k in range(3)
        ]
        rowmin = None
        for j in range(J):
            d0 = a[0] - t[j][0]
            d1 = a[1] - t[j][1]
            d2 = a[2] - t[j][2]
            acc = d0 * d0 + d1 * d1 + d2 * d2
            rowmin = acc if rowmin is None else jnp.minimum(rowmin, acc)
            colacc[j] = (
                acc if colacc[j] is None else jnp.minimum(colacc[j], acc)
            )
        d1s_ref[rs, :] = jnp.minimum(d1s_ref[rs, :], rowmin)

    for j in range(J):
        sl = pl.ds(m * TM + j * 128, 128)
        d2s_ref[:, sl] = jnp.minimum(d2s_ref[:, sl], colacc[j])

    @pl.when(m == num_m - 1)
    def _():
        d1_ref[0, 0, :] = jnp.min(d1s_ref[...], axis=1)

    @pl.when(n == num_n - 1)
    def _():
        d2_ref[0, 0, :] = jnp.min(d2s_ref[:, pl.ds(m * TM, TM)], axis=0)


@jax.jit
def kernel(xyz1, xyz2):
    B, N, _ = xyz1.shape
    _, M, _ = xyz2.shape
    x2t = jnp.transpose(xyz2, (0, 2, 1))

    grid = (B, N // TN, M // TM)
    dist1, dist2 = pl.pallas_call(
        _chamfer_body,
        grid=grid,
        in_specs=[
            pl.BlockSpec((1, TN, 3), lambda b, n, m: (b, n, 0)),
            pl.BlockSpec((1, 3, TM), lambda b, n, m: (b, 0, m)),
        ],
        out_specs=[
            pl.BlockSpec((1, 1, TN), lambda b, n, m: (b, 0, n)),
            pl.BlockSpec((1, 1, TM), lambda b, n, m: (b, 0, m)),
        ],
        out_shape=[
            jax.ShapeDtypeStruct((B, 1, N), jnp.float32),
            jax.ShapeDtypeStruct((B, 1, M), jnp.float32),
        ],
        scratch_shapes=[
            pltpu.VMEM((TN, 128), jnp.float32),
            pltpu.VMEM((8, M), jnp.float32),
        ],
        compiler_params=pltpu.CompilerParams(
            dimension_semantics=("arbitrary", "arbitrary", "arbitrary"),
        ),
    )(xyz1, x2t)
    return (dist1[:, 0, :], dist2[:, 0, :])


# speedup vs baseline: 1.8570x; 1.0230x over previous
import jax
import jax.numpy as jnp
from jax.experimental import pallas as pl
from jax.experimental.pallas import tpu as pltpu

TN = 512
TM = 1024
JH = 4


def _chamfer_body(x1_ref, x2t_ref, d1_ref, d2_ref, x1b_ref, d1s_ref, d2s_ref):
    n = pl.program_id(1)
    m = pl.program_id(2)
    num_n = pl.num_programs(1)
    num_m = pl.num_programs(2)
    J = TM // 128
    R = TN // 8

    @pl.when(m == 0)
    def _():
        for k in range(3):
            x1b_ref[k] = jnp.broadcast_to(x1_ref[0, :, k : k + 1], (TN, 128))
        d1s_ref[...] = jnp.full((TN, 128), jnp.inf, jnp.float32)

    @pl.when((n == 0) & (m == 0))
    def _():
        d2s_ref[...] = jnp.full(d2s_ref.shape, jnp.inf, jnp.float32)

    for j0 in range(0, J, JH):
        t = [
            [
                jnp.broadcast_to(
                    x2t_ref[0, k : k + 1, pl.ds((j0 + j) * 128, 128)], (8, 128)
                )
                for k in range(3)
            ]
            for j in range(JH)
        ]
        colacc = [None] * JH
        for r in range(R):
            rs = pl.ds(r * 8, 8)
            a = [x1b_ref[k, rs, :] for k in range(3)]
            rowmin = None
            for j in range(JH):
                d0 = a[0] - t[j][0]
                d1 = a[1] - t[j][1]
                d2 = a[2] - t[j][2]
                acc = d0 * d0 + d1 * d1 + d2 * d2
                rowmin = acc if rowmin is None else jnp.minimum(rowmin, acc)
                colacc[j] = (
                    acc if colacc[j] is None else jnp.minimum(colacc[j], acc)
                )
            d1s_ref[rs, :] = jnp.minimum(d1s_ref[rs, :], rowmin)

        for j in range(JH):
            sl = pl.ds(m * TM + (j0 + j) * 128, 128)
            d2s_ref[:, sl] = jnp.minimum(d2s_ref[:, sl], colacc[j])

    @pl.when(m == num_m - 1)
    def _():
        d1_ref[0, 0, :] = jnp.min(d1s_ref[...], axis=1)

    @pl.when(n == num_n - 1)
    def _():
        d2_ref[0, 0, :] = jnp.min(d2s_ref[:, pl.ds(m * TM, TM)], axis=0)


@jax.jit
def kernel(xyz1, xyz2):
    B, N, _ = xyz1.shape
    _, M, _ = xyz2.shape
    x2t = jnp.transpose(xyz2, (0, 2, 1))

    grid = (B, N // TN, M // TM)
    dist1, dist2 = pl.pallas_call(
        _chamfer_body,
        grid=grid,
        in_specs=[
            pl.BlockSpec((1, TN, 3), lambda b, n, m: (b, n, 0)),
            pl.BlockSpec((1, 3, TM), lambda b, n, m: (b, 0, m)),
        ],
        out_specs=[
            pl.BlockSpec((1, 1, TN), lambda b, n, m: (b, 0, n)),
            pl.BlockSpec((1, 1, TM), lambda b, n, m: (b, 0, m)),
        ],
        out_shape=[
            jax.ShapeDtypeStruct((B, 1, N), jnp.float32),
            jax.ShapeDtypeStruct((B, 1, M), jnp.float32),
        ],
        scratch_shapes=[
            pltpu.VMEM((3, TN, 128), jnp.float32),
            pltpu.VMEM((TN, 128), jnp.float32),
            pltpu.VMEM((8, M), jnp.float32),
        ],
        compiler_params=pltpu.CompilerParams(
            dimension_semantics=("arbitrary", "arbitrary", "arbitrary"),
        ),
    )(xyz1, x2t)
    return (dist1[:, 0, :], dist2[:, 0, :])
